# baseline (speedup 1.0000x reference)
import functools

import jax
import jax.numpy as jnp
import numpy as np
from jax import lax
from jax.experimental import pallas as pl
from jax.experimental.pallas import tpu as pltpu
from jax.experimental.pallas import tpu_sc as plsc

_B = 16384
_EMB = 16
_NCO = 13
_NCA = (5, 5, 5, 1)
_D = 29 * _EMB
_NC = 2
_NS = 16
_NW = _NC * _NS
_BPW = _B // _NW
_CHUNK = 128
_M = 512

_GRP_CHUNKS = tuple(nf * _BPW // _CHUNK for nf in _NCA)
_MAXC = max(_GRP_CHUNKS)

def _mk_rep(nf):
    r = np.zeros((nf, nf * _EMB), np.float32)
    for j in range(nf):
        r[j, j * _EMB:(j + 1) * _EMB] = 1.0
    return r

_R5 = _mk_rep(5)
_S13 = np.tile(np.eye(_EMB, dtype=np.float32), (13, 1))
_S5 = np.tile(np.eye(_EMB, dtype=np.float32), (5, 1))
_P13 = np.repeat(np.eye(13, dtype=np.float32), 14, axis=1)
_KPAT = np.tile(np.arange(14, dtype=np.float32), 13)


_MP = 2048


def _idx_prep(xi32):
    def body(xi_ref, o0, o1, o2, o3):
        x = xi_ref[...]
        o0[...] = x[:, 13:18]
        o1[...] = x[:, 18:23]
        o2[...] = x[:, 23:28]
        o3[...] = x[:, 28:29]

    return pl.pallas_call(
        body,
        grid=(_B // _MP,),
        in_specs=[pl.BlockSpec((_MP, 29), lambda i: (i, 0))],
        out_specs=[
            pl.BlockSpec((_MP, 5), lambda i: (i, 0)),
            pl.BlockSpec((_MP, 5), lambda i: (i, 0)),
            pl.BlockSpec((_MP, 5), lambda i: (i, 0)),
            pl.BlockSpec((_MP, 1), lambda i: (i, 0)),
        ],
        out_shape=[
            jax.ShapeDtypeStruct((_B, 5), jnp.int32),
            jax.ShapeDtypeStruct((_B, 5), jnp.int32),
            jax.ShapeDtypeStruct((_B, 5), jnp.int32),
            jax.ShapeDtypeStruct((_B, 1), jnp.int32),
        ],
    )(xi32)


def _sc_gather(idx_groups, tables):
    mesh = plsc.VectorSubcoreMesh(core_axis_name="c", subcore_axis_name="s")
    out_type = tuple(
        jax.ShapeDtypeStruct((_B * nf, _EMB), jnp.float32) for nf in _NCA
    )

    @functools.partial(
        pl.kernel,
        out_type=out_type,
        mesh=mesh,
        scratch_types=[
            pltpu.VMEM((_MAXC, _CHUNK), jnp.int32),
            pltpu.VMEM((_MAXC * _CHUNK, _EMB), jnp.float32),
            pltpu.SemaphoreType.DMA,
        ],
        compiler_params=pltpu.CompilerParams(use_tc_tiling_on_sc=False),
    )
    def gather_k(i0, i1, i2, i3, t0, t1, t2, t3,
                 o0, o1, o2, o3, idx_v, rows_v, sem):
        cid = lax.axis_index("c")
        sid = lax.axis_index("s")
        wid = sid * _NC + cid
        idxs = (i0, i1, i2, i3)
        tbls = (t0, t1, t2, t3)
        outs = (o0, o1, o2, o3)
        for g in range(4):
            nch = _GRP_CHUNKS[g]
            nf = _NCA[g]
            tbl = tbls[g]
            pltpu.sync_copy(idxs[g].at[wid], idx_v.at[pl.ds(0, nch)])

            def fire(it, carry, tbl=tbl):
                pltpu.async_copy(
                    tbl.at[idx_v.at[it]],
                    rows_v.at[pl.ds(it * _CHUNK, _CHUNK)],
                    sem,
                )
                return carry

            lax.fori_loop(0, nch, fire, 0)
            pltpu.make_async_copy(
                outs[g].at[pl.ds(0, nch * _CHUNK)],
                rows_v.at[pl.ds(0, nch * _CHUNK)],
                sem,
            ).wait()
            pltpu.sync_copy(rows_v.at[pl.ds(0, nch * _CHUNK)],
                            outs[g].at[pl.ds(wid * nf * _BPW, nf * _BPW)])

    return gather_k(*idx_groups, *tables)


def _tc_body(gt0, gt1, gt2, gt3, xi_co, xv,
             w1a, w1b, w1c, w1d, w1e, b1, w2, b2, w3, b3, w4, b4,
             p13, kpat, e2bd, fcot, r5, s13, s5, wf0, wfs, wfd, bfc, out):
    f32 = jnp.float32
    dot = functools.partial(jnp.dot, preferred_element_type=f32)
    xv_b = xv[...]
    xvco = xv_b[:, 0:13]

    xi_f = xi_co[...].astype(f32)
    oh = jnp.where(dot(xi_f, p13[...]) == kpat[...],
                   dot(xvco, p13[...]), 0.0)
    h_co = dot(oh, e2bd[...])
    y1 = dot(oh, fcot[...])

    h_t0 = gt0[...] * dot(xv_b[:, 13:18], r5[...])
    h_t1 = gt1[...] * dot(xv_b[:, 18:23], r5[...])
    h_t2 = gt2[...] * dot(xv_b[:, 23:28], r5[...])
    h_t3 = gt3[...] * xv_b[:, 28:29]

    summed = (dot(h_co, s13[...]) + dot(h_t0, s5[...]) + dot(h_t1, s5[...])
              + dot(h_t2, s5[...]) + h_t3)
    sq = (dot(h_co * h_co, s13[...]) + dot(h_t0 * h_t0, s5[...])
          + dot(h_t1 * h_t1, s5[...]) + dot(h_t2 * h_t2, s5[...])
          + h_t3 * h_t3)
    y2 = 0.5 * (summed * summed - sq)

    z = (dot(h_co, w1a[...]) + dot(h_t0, w1b[...]) + dot(h_t1, w1c[...])
         + dot(h_t2, w1d[...]) + dot(h_t3, w1e[...]) + b1[...])
    h1 = jnp.maximum(z, 0.0)
    h2 = jnp.maximum(dot(h1, w2[...]) + b2[...], 0.0)
    h3 = jnp.maximum(dot(h2, w3[...]) + b3[...], 0.0)
    yd = dot(h3, w4[...]) + b4[...]

    res = (y1 * wf0[0]
           + jnp.sum(y2 * wfs[...], axis=1, keepdims=True)
           + jnp.sum(yd * wfd[...], axis=1, keepdims=True)
           + bfc[0])
    out[...] = res


def _tc_forward(gt0, gt1, gt2, gt3, xi_co, xv,
                w1a, w1b, w1c, w1d, w1e, b1, w2, b2, w3, b3, w4, b4,
                e2bd, fcot, wf0, wfs, wfd, bfc):
    grid = (_B // _M,)

    def batch_spec(width):
        return pl.BlockSpec((_M, width), lambda i: (i, 0))

    def full_spec(shape):
        nd = len(shape)
        return pl.BlockSpec(shape, lambda i: (0,) * nd)

    smem = pl.BlockSpec(memory_space=pltpu.SMEM)

    in_specs = [
        batch_spec(80), batch_spec(80), batch_spec(80), batch_spec(_EMB),
        batch_spec(13),
        batch_spec(29),
        full_spec((208, _D)), full_spec((80, _D)), full_spec((80, _D)),
        full_spec((80, _D)), full_spec((16, _D)), full_spec((1, _D)),
        full_spec((_D, _D)), full_spec((1, _D)),
        full_spec((_D, _D)), full_spec((1, _D)),
        full_spec((_D, _EMB)), full_spec((1, _EMB)),
        full_spec((13, 182)), full_spec((1, 182)),
        full_spec((182, 208)), full_spec((182, 1)),
        full_spec((5, 80)),
        full_spec((208, _EMB)), full_spec((80, _EMB)),
        smem,
        full_spec((1, _EMB)), full_spec((1, _EMB)),
        smem,
    ]

    return pl.pallas_call(
        _tc_body,
        grid=grid,
        in_specs=in_specs,
        out_specs=pl.BlockSpec((_M, 1), lambda i: (i, 0)),
        out_shape=jax.ShapeDtypeStruct((_B, 1), jnp.float32),
        compiler_params=pltpu.CompilerParams(
            dimension_semantics=("arbitrary",),
        ),
    )(gt0, gt1, gt2, gt3, xi_co, xv,
      w1a, w1b, w1c, w1d, w1e, b1, w2, b2, w3, b3, w4, b4,
      jnp.asarray(_P13), jnp.asarray(_KPAT).reshape(1, 182),
      e2bd, fcot,
      jnp.asarray(_R5), jnp.asarray(_S13), jnp.asarray(_S5),
      wf0, wfs, wfd, bfc)


def kernel(xi, xv, first_co_emb, first_ca_emb0, first_ca_emb1, first_ca_emb2,
           first_ca_emb3, second_co_emb, second_ca_emb0, second_ca_emb1,
           second_ca_emb2, second_ca_emb3, W1, b1, W2, b2, W3, b3, W4, b4,
           Wfc, bfc):
    del first_ca_emb0, first_ca_emb1, first_ca_emb2, first_ca_emb3
    xi32 = xi.astype(jnp.int32)
    idx_flat = _idx_prep(xi32)
    idx_groups = [a.reshape(_NW, -1, _CHUNK) for a in idx_flat]
    tables = [second_ca_emb0, second_ca_emb1, second_ca_emb2, second_ca_emb3]
    g = _sc_gather(idx_groups, tables)
    gt0 = g[0].reshape(_B, 80)
    gt1 = g[1].reshape(_B, 80)
    gt2 = g[2].reshape(_B, 80)
    gt3 = g[3].reshape(_B, _EMB)

    w1a = W1[0:208]
    w1b = W1[208:288]
    w1c = W1[288:368]
    w1d = W1[368:448]
    w1e = W1[448:464]
    wf0 = Wfc[0, :]
    wfs = Wfc[1:17, 0].reshape(1, _EMB)
    wfd = Wfc[17:33, 0].reshape(1, _EMB)
    e2bd = jnp.kron(jnp.eye(13, dtype=jnp.float32), second_co_emb)
    fcot = jnp.tile(first_co_emb, (13, 1))

    return _tc_forward(
        gt0, gt1, gt2, gt3, xi32[:, 0:13], xv,
        w1a, w1b, w1c, w1d, w1e, b1.reshape(1, _D),
        W2, b2.reshape(1, _D), W3, b3.reshape(1, _D),
        W4, b4.reshape(1, _EMB),
        e2bd, fcot, wf0, wfs, wfd, bfc)


# speedup vs baseline: 1.0136x; 1.0136x over previous
import functools

import jax
import jax.numpy as jnp
import numpy as np
from jax import lax
from jax.experimental import pallas as pl
from jax.experimental.pallas import tpu as pltpu
from jax.experimental.pallas import tpu_sc as plsc

_B = 16384
_EMB = 16
_NCO = 13
_NCA = (5, 5, 5, 1)
_D = 29 * _EMB
_NC = 2
_NS = 16
_NW = _NC * _NS
_BPW = _B // _NW
_CHUNK = 128
_M = 512

_GRP_CHUNKS = tuple(nf * _BPW // _CHUNK for nf in _NCA)
_MAXC = max(_GRP_CHUNKS)

def _mk_rep(nf):
    r = np.zeros((nf, nf * _EMB), np.float32)
    for j in range(nf):
        r[j, j * _EMB:(j + 1) * _EMB] = 1.0
    return r

_R5 = _mk_rep(5)
_S13 = np.tile(np.eye(_EMB, dtype=np.float32), (13, 1))
_S5 = np.tile(np.eye(_EMB, dtype=np.float32), (5, 1))
_P13 = np.repeat(np.eye(13, dtype=np.float32), 14, axis=1)
_KPAT = np.tile(np.arange(14, dtype=np.float32), 13)


_MP = 2048


def _idx_prep(xi32):
    def body(xi_ref, o0, o1, o2, o3):
        x = xi_ref[...]
        o0[...] = x[:, 13:18]
        o1[...] = x[:, 18:23]
        o2[...] = x[:, 23:28]
        o3[...] = x[:, 28:29]

    return pl.pallas_call(
        body,
        grid=(_B // _MP,),
        in_specs=[pl.BlockSpec((_MP, 29), lambda i: (i, 0))],
        out_specs=[
            pl.BlockSpec((_MP, 5), lambda i: (i, 0)),
            pl.BlockSpec((_MP, 5), lambda i: (i, 0)),
            pl.BlockSpec((_MP, 5), lambda i: (i, 0)),
            pl.BlockSpec((_MP, 1), lambda i: (i, 0)),
        ],
        out_shape=[
            jax.ShapeDtypeStruct((_B, 5), jnp.int32),
            jax.ShapeDtypeStruct((_B, 5), jnp.int32),
            jax.ShapeDtypeStruct((_B, 5), jnp.int32),
            jax.ShapeDtypeStruct((_B, 1), jnp.int32),
        ],
    )(xi32)


def _sc_gather(idx_groups, tables):
    mesh = plsc.VectorSubcoreMesh(core_axis_name="c", subcore_axis_name="s")
    out_type = tuple(
        jax.ShapeDtypeStruct((_B * nf, _EMB), jnp.float32) for nf in _NCA
    )

    @functools.partial(
        pl.kernel,
        out_type=out_type,
        mesh=mesh,
        scratch_types=[
            pltpu.VMEM((_MAXC, _CHUNK), jnp.int32),
            pltpu.VMEM((_MAXC * _CHUNK, _EMB), jnp.float32),
            pltpu.SemaphoreType.DMA,
        ],
        compiler_params=pltpu.CompilerParams(use_tc_tiling_on_sc=False),
    )
    def gather_k(i0, i1, i2, i3, t0, t1, t2, t3,
                 o0, o1, o2, o3, idx_v, rows_v, sem):
        cid = lax.axis_index("c")
        sid = lax.axis_index("s")
        wid = sid * _NC + cid
        idxs = (i0, i1, i2, i3)
        tbls = (t0, t1, t2, t3)
        outs = (o0, o1, o2, o3)
        for g in range(4):
            nch = _GRP_CHUNKS[g]
            nf = _NCA[g]
            tbl = tbls[g]
            pltpu.sync_copy(idxs[g].at[wid], idx_v.at[pl.ds(0, nch)])

            def fire(it, carry, tbl=tbl):
                pltpu.async_copy(
                    tbl.at[idx_v.at[it]],
                    rows_v.at[pl.ds(it * _CHUNK, _CHUNK)],
                    sem,
                )
                return carry

            lax.fori_loop(0, nch, fire, 0)
            pltpu.make_async_copy(
                outs[g].at[pl.ds(0, nch * _CHUNK)],
                rows_v.at[pl.ds(0, nch * _CHUNK)],
                sem,
            ).wait()
            pltpu.sync_copy(rows_v.at[pl.ds(0, nch * _CHUNK)],
                            outs[g].at[pl.ds(wid * nf * _BPW, nf * _BPW)])

    return gather_k(*idx_groups, *tables)


def _tc_body(gt0, gt1, gt2, gt3, xi_co, xv,
             w1a, w1b, w1c, w1d, w1e, b1, w2, b2, w3, b3, w4, b4,
             p13, kpat, e2bd, fcot, r5, s13, s5, wf0, wfs, wfd, bfc, out):
    f32 = jnp.float32
    dot = functools.partial(jnp.dot, preferred_element_type=f32)
    xv_b = xv[...]
    xvco = xv_b[:, 0:13]

    xi_f = xi_co[...].astype(f32)
    oh = jnp.where(dot(xi_f, p13[...]) == kpat[...],
                   dot(xvco, p13[...]), 0.0)
    h_co = dot(oh, e2bd[...])
    y1 = dot(oh, fcot[...])

    h_t0 = gt0[...] * dot(xv_b[:, 13:18], r5[...])
    h_t1 = gt1[...] * dot(xv_b[:, 18:23], r5[...])
    h_t2 = gt2[...] * dot(xv_b[:, 23:28], r5[...])
    h_t3 = gt3[...] * xv_b[:, 28:29]

    summed = (dot(h_co, s13[...]) + dot(h_t0, s5[...]) + dot(h_t1, s5[...])
              + dot(h_t2, s5[...]) + h_t3)
    sq = (dot(h_co * h_co, s13[...]) + dot(h_t0 * h_t0, s5[...])
          + dot(h_t1 * h_t1, s5[...]) + dot(h_t2 * h_t2, s5[...])
          + h_t3 * h_t3)
    y2 = 0.5 * (summed * summed - sq)

    bf = jnp.bfloat16
    z = (dot(h_co.astype(bf), w1a[...]) + dot(h_t0.astype(bf), w1b[...])
         + dot(h_t1.astype(bf), w1c[...]) + dot(h_t2.astype(bf), w1d[...])
         + dot(h_t3.astype(bf), w1e[...]) + b1[...])
    h1 = jnp.maximum(z, 0.0)
    h2 = jnp.maximum(dot(h1.astype(bf), w2[...]) + b2[...], 0.0)
    h3 = jnp.maximum(dot(h2.astype(bf), w3[...]) + b3[...], 0.0)
    yd = dot(h3.astype(bf), w4[...]) + b4[...]

    res = (y1 * wf0[0]
           + jnp.sum(y2 * wfs[...], axis=1, keepdims=True)
           + jnp.sum(yd * wfd[...], axis=1, keepdims=True)
           + bfc[0])
    out[...] = res


def _tc_forward(gt0, gt1, gt2, gt3, xi_co, xv,
                w1a, w1b, w1c, w1d, w1e, b1, w2, b2, w3, b3, w4, b4,
                e2bd, fcot, wf0, wfs, wfd, bfc):
    grid = (_B // _M,)

    def batch_spec(width):
        return pl.BlockSpec((_M, width), lambda i: (i, 0))

    def full_spec(shape):
        nd = len(shape)
        return pl.BlockSpec(shape, lambda i: (0,) * nd)

    smem = pl.BlockSpec(memory_space=pltpu.SMEM)

    in_specs = [
        batch_spec(80), batch_spec(80), batch_spec(80), batch_spec(_EMB),
        batch_spec(13),
        batch_spec(29),
        full_spec((208, _D)), full_spec((80, _D)), full_spec((80, _D)),
        full_spec((80, _D)), full_spec((16, _D)), full_spec((1, _D)),
        full_spec((_D, _D)), full_spec((1, _D)),
        full_spec((_D, _D)), full_spec((1, _D)),
        full_spec((_D, _EMB)), full_spec((1, _EMB)),

        full_spec((13, 182)), full_spec((1, 182)),
        full_spec((182, 208)), full_spec((182, 1)),
        full_spec((5, 80)),
        full_spec((208, _EMB)), full_spec((80, _EMB)),
        smem,
        full_spec((1, _EMB)), full_spec((1, _EMB)),
        smem,
    ]

    return pl.pallas_call(
        _tc_body,
        grid=grid,
        in_specs=in_specs,
        out_specs=pl.BlockSpec((_M, 1), lambda i: (i, 0)),
        out_shape=jax.ShapeDtypeStruct((_B, 1), jnp.float32),
        compiler_params=pltpu.CompilerParams(
            dimension_semantics=("arbitrary",),
        ),
    )(gt0, gt1, gt2, gt3, xi_co, xv,
      w1a, w1b, w1c, w1d, w1e, b1, w2, b2, w3, b3, w4, b4,
      jnp.asarray(_P13), jnp.asarray(_KPAT).reshape(1, 182),
      e2bd, fcot,
      jnp.asarray(_R5), jnp.asarray(_S13), jnp.asarray(_S5),
      wf0, wfs, wfd, bfc)


def kernel(xi, xv, first_co_emb, first_ca_emb0, first_ca_emb1, first_ca_emb2,
           first_ca_emb3, second_co_emb, second_ca_emb0, second_ca_emb1,
           second_ca_emb2, second_ca_emb3, W1, b1, W2, b2, W3, b3, W4, b4,
           Wfc, bfc):
    del first_ca_emb0, first_ca_emb1, first_ca_emb2, first_ca_emb3
    xi32 = xi.astype(jnp.int32)
    idx_flat = _idx_prep(xi32)
    idx_groups = [a.reshape(_NW, -1, _CHUNK) for a in idx_flat]
    def _compact(t):
        v = t.shape[0]
        return lax.optimization_barrier(t.reshape(v // 8, 128)).reshape(v, 16)

    tables = [_compact(second_ca_emb0), _compact(second_ca_emb1),
              _compact(second_ca_emb2), second_ca_emb3]
    g = _sc_gather(idx_groups, tables)
    gt0 = g[0].reshape(_B, 80)
    gt1 = g[1].reshape(_B, 80)
    gt2 = g[2].reshape(_B, 80)
    gt3 = g[3].reshape(_B, _EMB)

    w1bf = W1.astype(jnp.bfloat16)
    w1a = w1bf[0:208]
    w1b = w1bf[208:288]
    w1c = w1bf[288:368]
    w1d = w1bf[368:448]
    w1e = w1bf[448:464]
    wf0 = Wfc[0, :]
    wfs = Wfc[1:17, 0].reshape(1, _EMB)
    wfd = Wfc[17:33, 0].reshape(1, _EMB)
    e2bd = jnp.kron(jnp.eye(13, dtype=jnp.float32), second_co_emb)
    fcot = jnp.tile(first_co_emb, (13, 1))

    return _tc_forward(
        gt0, gt1, gt2, gt3, xi32[:, 0:13], xv,
        w1a, w1b, w1c, w1d, w1e, b1.reshape(1, _D),
        W2.astype(jnp.bfloat16), b2.reshape(1, _D),
        W3.astype(jnp.bfloat16), b3.reshape(1, _D),
        W4.astype(jnp.bfloat16), b4.reshape(1, _EMB),
        e2bd, fcot, wf0, wfs, wfd, bfc)


# speedup vs baseline: 1.0228x; 1.0090x over previous
import functools

import jax
import jax.numpy as jnp
import numpy as np
from jax import lax
from jax.experimental import pallas as pl
from jax.experimental.pallas import tpu as pltpu
from jax.experimental.pallas import tpu_sc as plsc

_B = 16384
_EMB = 16
_NCO = 13
_NCA = (5, 5, 5, 1)
_D = 29 * _EMB
_NC = 2
_NS = 16
_NW = _NC * _NS
_BPW = _B // _NW
_CHUNK = 128
_M = 1024

_GRP_CHUNKS = tuple(nf * _BPW // _CHUNK for nf in _NCA)
_MAXC = max(_GRP_CHUNKS)

def _mk_rep(nf):
    r = np.zeros((nf, nf * _EMB), np.float32)
    for j in range(nf):
        r[j, j * _EMB:(j + 1) * _EMB] = 1.0
    return r

_R5 = _mk_rep(5)
_S13 = np.tile(np.eye(_EMB, dtype=np.float32), (13, 1))
_S5 = np.tile(np.eye(_EMB, dtype=np.float32), (5, 1))
_P13 = np.repeat(np.eye(13, dtype=np.float32), 14, axis=1)
_KPAT = np.tile(np.arange(14, dtype=np.float32), 13)


_MP = 2048


def _idx_prep(xi32):
    def body(xi_ref, o0, o1, o2, o3):
        x = xi_ref[...]
        o0[...] = x[:, 13:18]
        o1[...] = x[:, 18:23]
        o2[...] = x[:, 23:28]
        o3[...] = x[:, 28:29]

    return pl.pallas_call(
        body,
        grid=(_B // _MP,),
        in_specs=[pl.BlockSpec((_MP, 29), lambda i: (i, 0))],
        out_specs=[
            pl.BlockSpec((_MP, 5), lambda i: (i, 0)),
            pl.BlockSpec((_MP, 5), lambda i: (i, 0)),
            pl.BlockSpec((_MP, 5), lambda i: (i, 0)),
            pl.BlockSpec((_MP, 1), lambda i: (i, 0)),
        ],
        out_shape=[
            jax.ShapeDtypeStruct((_B, 5), jnp.int32),
            jax.ShapeDtypeStruct((_B, 5), jnp.int32),
            jax.ShapeDtypeStruct((_B, 5), jnp.int32),
            jax.ShapeDtypeStruct((_B, 1), jnp.int32),
        ],
    )(xi32)


def _sc_gather(idx_groups, tables):
    mesh = plsc.VectorSubcoreMesh(core_axis_name="c", subcore_axis_name="s")
    out_type = tuple(
        jax.ShapeDtypeStruct((_B * nf, _EMB), jnp.float32) for nf in _NCA
    )

    @functools.partial(
        pl.kernel,
        out_type=out_type,
        mesh=mesh,
        scratch_types=[
            pltpu.VMEM((_MAXC, _CHUNK), jnp.int32),
            pltpu.VMEM((_MAXC * _CHUNK, _EMB), jnp.float32),
            pltpu.SemaphoreType.DMA,
        ],
        compiler_params=pltpu.CompilerParams(use_tc_tiling_on_sc=False),
    )
    def gather_k(i0, i1, i2, i3, t0, t1, t2, t3,
                 o0, o1, o2, o3, idx_v, rows_v, sem):
        cid = lax.axis_index("c")
        sid = lax.axis_index("s")
        wid = sid * _NC + cid
        idxs = (i0, i1, i2, i3)
        tbls = (t0, t1, t2, t3)
        outs = (o0, o1, o2, o3)
        for g in range(4):
            nch = _GRP_CHUNKS[g]
            nf = _NCA[g]
            tbl = tbls[g]
            pltpu.sync_copy(idxs[g].at[wid], idx_v.at[pl.ds(0, nch)])

            def fire(it, carry, tbl=tbl):
                pltpu.async_copy(
                    tbl.at[idx_v.at[it]],
                    rows_v.at[pl.ds(it * _CHUNK, _CHUNK)],
                    sem,
                )
                return carry

            lax.fori_loop(0, nch, fire, 0)
            pltpu.make_async_copy(
                outs[g].at[pl.ds(0, nch * _CHUNK)],
                rows_v.at[pl.ds(0, nch * _CHUNK)],
                sem,
            ).wait()
            pltpu.sync_copy(rows_v.at[pl.ds(0, nch * _CHUNK)],
                            outs[g].at[pl.ds(wid * nf * _BPW, nf * _BPW)])

    return gather_k(*idx_groups, *tables)


def _tc_body(gt0, gt1, gt2, gt3, xi_co, xv,
             w1a, w1b, w1c, w1d, w1e, b1, w2, b2, w3, b3, w4, b4,
             p13, kpat, e2bd, fcot, r5, s13, s5, wf0, wfs, wfd, bfc, out):
    f32 = jnp.float32
    dot = functools.partial(jnp.dot, preferred_element_type=f32)
    xv_b = xv[...]
    xvco = xv_b[:, 0:13]

    xi_f = xi_co[...].astype(f32)
    oh = jnp.where(dot(xi_f, p13[...]) == kpat[...],
                   dot(xvco, p13[...]), 0.0)
    h_co = dot(oh, e2bd[...])
    y1 = dot(oh, fcot[...])

    h_t0 = gt0[...] * dot(xv_b[:, 13:18], r5[...])
    h_t1 = gt1[...] * dot(xv_b[:, 18:23], r5[...])
    h_t2 = gt2[...] * dot(xv_b[:, 23:28], r5[...])
    h_t3 = gt3[...] * xv_b[:, 28:29]

    summed = (dot(h_co, s13[...]) + dot(h_t0, s5[...]) + dot(h_t1, s5[...])
              + dot(h_t2, s5[...]) + h_t3)
    sq = (dot(h_co * h_co, s13[...]) + dot(h_t0 * h_t0, s5[...])
          + dot(h_t1 * h_t1, s5[...]) + dot(h_t2 * h_t2, s5[...])
          + h_t3 * h_t3)
    y2 = 0.5 * (summed * summed - sq)

    bf = jnp.bfloat16
    z = (dot(h_co.astype(bf), w1a[...]) + dot(h_t0.astype(bf), w1b[...])
         + dot(h_t1.astype(bf), w1c[...]) + dot(h_t2.astype(bf), w1d[...])
         + dot(h_t3.astype(bf), w1e[...]) + b1[...])
    h1 = jnp.maximum(z, 0.0)
    h2 = jnp.maximum(dot(h1.astype(bf), w2[...]) + b2[...], 0.0)
    h3 = jnp.maximum(dot(h2.astype(bf), w3[...]) + b3[...], 0.0)
    yd = dot(h3.astype(bf), w4[...]) + b4[...]

    res = (y1 * wf0[0]
           + jnp.sum(y2 * wfs[...], axis=1, keepdims=True)
           + jnp.sum(yd * wfd[...], axis=1, keepdims=True)
           + bfc[0])
    out[...] = res


def _tc_forward(gt0, gt1, gt2, gt3, xi_co, xv,
                w1a, w1b, w1c, w1d, w1e, b1, w2, b2, w3, b3, w4, b4,
                e2bd, fcot, wf0, wfs, wfd, bfc):
    grid = (_B // _M,)

    def batch_spec(width):
        return pl.BlockSpec((_M, width), lambda i: (i, 0))

    def full_spec(shape):
        nd = len(shape)
        return pl.BlockSpec(shape, lambda i: (0,) * nd)

    smem = pl.BlockSpec(memory_space=pltpu.SMEM)

    in_specs = [
        batch_spec(80), batch_spec(80), batch_spec(80), batch_spec(_EMB),
        batch_spec(13),
        batch_spec(29),
        full_spec((208, _D)), full_spec((80, _D)), full_spec((80, _D)),
        full_spec((80, _D)), full_spec((16, _D)), full_spec((1, _D)),
        full_spec((_D, _D)), full_spec((1, _D)),
        full_spec((_D, _D)), full_spec((1, _D)),
        full_spec((_D, _EMB)), full_spec((1, _EMB)),

        full_spec((13, 182)), full_spec((1, 182)),
        full_spec((182, 208)), full_spec((182, 1)),
        full_spec((5, 80)),
        full_spec((208, _EMB)), full_spec((80, _EMB)),
        smem,
        full_spec((1, _EMB)), full_spec((1, _EMB)),
        smem,
    ]

    return pl.pallas_call(
        _tc_body,
        grid=grid,
        in_specs=in_specs,
        out_specs=pl.BlockSpec((_M, 1), lambda i: (i, 0)),
        out_shape=jax.ShapeDtypeStruct((_B, 1), jnp.float32),
        compiler_params=pltpu.CompilerParams(
            dimension_semantics=("arbitrary",),
        ),
    )(gt0, gt1, gt2, gt3, xi_co, xv,
      w1a, w1b, w1c, w1d, w1e, b1, w2, b2, w3, b3, w4, b4,
      jnp.asarray(_P13), jnp.asarray(_KPAT).reshape(1, 182),
      e2bd, fcot,
      jnp.asarray(_R5), jnp.asarray(_S13), jnp.asarray(_S5),
      wf0, wfs, wfd, bfc)


def kernel(xi, xv, first_co_emb, first_ca_emb0, first_ca_emb1, first_ca_emb2,
           first_ca_emb3, second_co_emb, second_ca_emb0, second_ca_emb1,
           second_ca_emb2, second_ca_emb3, W1, b1, W2, b2, W3, b3, W4, b4,
           Wfc, bfc):
    del first_ca_emb0, first_ca_emb1, first_ca_emb2, first_ca_emb3
    xi32 = xi.astype(jnp.int32)
    idx_flat = _idx_prep(xi32)
    idx_groups = [a.reshape(_NW, -1, _CHUNK) for a in idx_flat]
    tables = [second_ca_emb0, second_ca_emb1, second_ca_emb2, second_ca_emb3]
    g = _sc_gather(idx_groups, tables)
    gt0 = g[0].reshape(_B, 80)
    gt1 = g[1].reshape(_B, 80)
    gt2 = g[2].reshape(_B, 80)
    gt3 = g[3].reshape(_B, _EMB)

    w1bf = W1.astype(jnp.bfloat16)
    w1a = w1bf[0:208]
    w1b = w1bf[208:288]
    w1c = w1bf[288:368]
    w1d = w1bf[368:448]
    w1e = w1bf[448:464]
    wf0 = Wfc[0, :]
    wfs = Wfc[1:17, 0].reshape(1, _EMB)
    wfd = Wfc[17:33, 0].reshape(1, _EMB)
    e2bd = jnp.kron(jnp.eye(13, dtype=jnp.float32), second_co_emb)
    fcot = jnp.tile(first_co_emb, (13, 1))

    return _tc_forward(
        gt0, gt1, gt2, gt3, xi32[:, 0:13], xv,
        w1a, w1b, w1c, w1d, w1e, b1.reshape(1, _D),
        W2.astype(jnp.bfloat16), b2.reshape(1, _D),
        W3.astype(jnp.bfloat16), b3.reshape(1, _D),
        W4.astype(jnp.bfloat16), b4.reshape(1, _EMB),
        e2bd, fcot, wf0, wfs, wfd, bfc)
